# Initial kernel scaffold; baseline (speedup 1.0000x reference)
import jax
import jax.numpy as jnp
from jax.experimental import pallas as pl


def kernel(x, edge_index, batch, W1_rel, W1_root, b1, W2_rel, W2_root, b2, Wfc, bfc):
    raise NotImplementedError("write your pallas kernel here")


# speedup vs baseline: 7.9839x; 7.9839x over previous
import jax
import jax.numpy as jnp
from jax import lax
from jax.experimental import pallas as pl
from jax.experimental.pallas import tpu as pltpu
from jax.experimental.pallas import tpu_sc as plsc

N = 10000
E = 320000
D = 128
G = 16
FC_OUT = 2048

NC, NS, L = 2, 16, 16
NW = NC * NS
N_PAD = 10112
EB = 64
TILE_EDGES = 10240
E_PAD = TILE_EDGES * NW
ROWS_PER_TILE = N_PAD // NS
FLAT_W = N_PAD * G
FPT = FLAT_W // NS

BR = 1264
NBLK = N_PAD // BR


def _sc_agg_body(x_hbm, src_hbm, dst_hbm, batch_hbm, zf_hbm, zw_hbm,
                 agg_hbm, w_hbm,
                 acc, waccf, batch_l, src_b, dst_b, fi_b, ones_b, rows_b, sem):
  c = lax.axis_index("c")
  s = lax.axis_index("s")
  r0 = s * ROWS_PER_TILE
  f0 = s * FPT
  pltpu.sync_copy(zf_hbm.at[pl.ds(r0, ROWS_PER_TILE)],
                  acc.at[pl.ds(r0, ROWS_PER_TILE)])
  pltpu.sync_copy(zw_hbm.at[pl.ds(f0, FPT)], waccf.at[pl.ds(f0, FPT)])
  pltpu.sync_copy(batch_hbm, batch_l)
  for k in range(EB // L):
    ones_b[pl.ds(k * L, L)] = jnp.ones((L,), jnp.float32)
  plsc.subcore_barrier()

  ebase = (c * NS + s) * TILE_EDGES

  def body(i, carry):
    off = ebase + i * EB
    pltpu.sync_copy(src_hbm.at[pl.ds(off, EB)], src_b)
    pltpu.sync_copy(dst_hbm.at[pl.ds(off, EB)], dst_b)
    pltpu.async_copy(x_hbm.at[src_b], rows_b, sem).wait()
    pltpu.sync_copy(rows_b, acc.at[dst_b], add=True)
    for k in range(EB // L):
      sv = src_b[pl.ds(k * L, L)]
      dv = dst_b[pl.ds(k * L, L)]
      gv = plsc.load_gather(batch_l, [dv])
      fi_b[pl.ds(k * L, L)] = sv * G + gv
    pltpu.sync_copy(ones_b, waccf.at[fi_b], add=True)
    return carry

  lax.fori_loop(0, TILE_EDGES // EB, body, 0)
  plsc.subcore_barrier()
  pltpu.sync_copy(acc.at[pl.ds(r0, ROWS_PER_TILE)],
                  agg_hbm.at[pl.ds(c * N_PAD + r0, ROWS_PER_TILE)])
  pltpu.sync_copy(waccf.at[pl.ds(f0, FPT)],
                  w_hbm.at[pl.ds(c * FLAT_W + f0, FPT)])


@jax.jit
def _sc_agg(x_pad, src_pad, dst_pad, batch_sc, zf, zwf):
  mesh = plsc.VectorSubcoreMesh(core_axis_name="c", subcore_axis_name="s",
                                num_cores=NC, num_subcores=NS)
  return pl.kernel(
      _sc_agg_body,
      out_type=(jax.ShapeDtypeStruct((NC * N_PAD, D), jnp.float32),
                jax.ShapeDtypeStruct((NC * FLAT_W,), jnp.float32)),
      mesh=mesh,
      compiler_params=pltpu.CompilerParams(needs_layout_passes=False),
      scratch_types=[
          pltpu.VMEM_SHARED((N_PAD, D), jnp.float32),
          pltpu.VMEM_SHARED((FLAT_W,), jnp.float32),
          pltpu.VMEM((N_PAD,), jnp.int32),
          pltpu.VMEM((EB,), jnp.int32),
          pltpu.VMEM((EB,), jnp.int32),
          pltpu.VMEM((EB,), jnp.int32),
          pltpu.VMEM((EB,), jnp.float32),
          pltpu.VMEM((EB, D), jnp.float32),
          pltpu.SemaphoreType.DMA,
      ],
  )(x_pad, src_pad, dst_pad, batch_sc, zf, zwf)


def _tc_body(agg_ref, w_ref, x_ref, batch_ref,
             w1r_ref, w1o_ref, b1_ref, w2r_ref, w2o_ref, b2_ref,
             wfc_ref, bfc_ref, out_ref, a_acc, b_acc, c_acc):
  i = pl.program_id(0)

  @pl.when(i == 0)
  def _init():
    a_acc[...] = jnp.zeros_like(a_acc)
    b_acc[...] = jnp.zeros_like(b_acc)
    c_acc[...] = jnp.zeros_like(c_acc)

  agg = agg_ref[0] + agg_ref[1]
  h = jnp.dot(agg, w1r_ref[...], preferred_element_type=jnp.float32)
  h += jnp.dot(x_ref[...], w1o_ref[...], preferred_element_type=jnp.float32)
  h = jnp.maximum(h + b1_ref[...], 0.0)

  rows = i * BR + lax.broadcasted_iota(jnp.int32, (BR, 1), 0)
  wm = jnp.where(rows < N, w_ref[0] + w_ref[1], 0.0)
  m = (batch_ref[...] ==
       lax.broadcasted_iota(jnp.int32, (BR, G), 1)).astype(jnp.float32)

  cdims = (((0,), (0,)), ((), ()))
  a_acc[...] += lax.dot_general(wm, h, cdims,
                                preferred_element_type=jnp.float32)
  b_acc[...] += lax.dot_general(m, h, cdims,
                                preferred_element_type=jnp.float32)
  c_acc[...] += lax.dot_general(m, jnp.ones((BR, D), jnp.float32), cdims,
                                preferred_element_type=jnp.float32)

  @pl.when(i == NBLK - 1)
  def _final():
    num = jnp.dot(a_acc[...], w2r_ref[...], preferred_element_type=jnp.float32)
    num += jnp.dot(b_acc[...], w2o_ref[...], preferred_element_type=jnp.float32)
    c = c_acc[...]
    pooled = (num + c * b2_ref[...]) / jnp.maximum(c, 1.0)
    out = jnp.dot(pooled, wfc_ref[...], preferred_element_type=jnp.float32)
    out_ref[...] = out + bfc_ref[...]


@jax.jit
def _tc_dense(agg_p, w_p, x_pad, batch_tc, w1rT, w1oT, b1, w2rT, w2oT, b2,
              wfcT, bfc):
  full = lambda shape: pl.BlockSpec(shape, lambda i: (0,) * len(shape))
  return pl.pallas_call(
      _tc_body,
      grid=(NBLK,),
      in_specs=[
          pl.BlockSpec((NC, BR, D), lambda i: (0, i, 0)),
          pl.BlockSpec((NC, BR, G), lambda i: (0, i, 0)),
          pl.BlockSpec((BR, D), lambda i: (i, 0)),
          pl.BlockSpec((BR, 1), lambda i: (i, 0)),
          full((D, D)), full((D, D)), full((1, D)),
          full((D, D)), full((D, D)), full((1, D)),
          full((D, FC_OUT)), full((1, FC_OUT)),
      ],
      out_specs=pl.BlockSpec((G, FC_OUT), lambda i: (0, 0)),
      out_shape=jax.ShapeDtypeStruct((G, FC_OUT), jnp.float32),
      scratch_shapes=[
          pltpu.VMEM((G, D), jnp.float32),
          pltpu.VMEM((G, D), jnp.float32),
          pltpu.VMEM((G, D), jnp.float32),
      ],
  )(agg_p, w_p, x_pad, batch_tc, w1rT, w1oT, b1, w2rT, w2oT, b2, wfcT, bfc)


def kernel(x, edge_index, batch, W1_rel, W1_root, b1, W2_rel, W2_root, b2,
           Wfc, bfc):
  src = edge_index[0].astype(jnp.int32)
  dst = edge_index[1].astype(jnp.int32)
  batch = batch.astype(jnp.int32)

  x_pad = jnp.zeros((N_PAD, D), jnp.float32).at[:N].set(x)
  pad_idx = N + (jnp.arange(E_PAD - E, dtype=jnp.int32) % (N_PAD - N))
  src_pad = jnp.concatenate([src, pad_idx])
  dst_pad = jnp.concatenate([dst, pad_idx])
  batch_sc = jnp.zeros((N_PAD,), jnp.int32).at[:N].set(batch)
  batch_tc = jnp.full((N_PAD, 1), G, jnp.int32).at[:N, 0].set(batch)
  zf = jnp.zeros((N_PAD, D), jnp.float32)
  zwf = jnp.zeros((FLAT_W,), jnp.float32)

  agg_f, w_f = _sc_agg(x_pad, src_pad, dst_pad, batch_sc, zf, zwf)
  agg_p = agg_f.reshape(NC, N_PAD, D)
  w_p = w_f.reshape(NC, N_PAD, G)
  out = _tc_dense(agg_p, w_p, x_pad, batch_tc,
                  W1_rel.T, W1_root.T, b1[None], W2_rel.T, W2_root.T,
                  b2[None], Wfc.T, bfc[None])
  return out[None]


# speedup vs baseline: 13.7478x; 1.7219x over previous
import jax
import jax.numpy as jnp
from jax import lax
from jax.experimental import pallas as pl
from jax.experimental.pallas import tpu as pltpu
from jax.experimental.pallas import tpu_sc as plsc

N = 10000
E = 320000
D = 128
G = 16
FC_OUT = 2048

NC, NS, L = 2, 16, 16
NW = NC * NS
N_PAD = 10112
EB = 64
TILE_EDGES = 10240
E_PAD = TILE_EDGES * NW
ROWS_PER_TILE = N_PAD // NS
FLAT_W = N_PAD * G
FPT = FLAT_W // NS

BR = 1264
NBLK = N_PAD // BR


def _sc_agg_body(x_hbm, src_hbm, dst_hbm, batchp_hbm, zf_hbm, zw_hbm,
                 agg_hbm, w_hbm,
                 acc, waccf, batch_p,
                 src_b0, src_b1, dst_b0, dst_b1, dsts0, dsts1,
                 fi_b, ones_b, rows_b0, rows_b1,
                 gsem0, gsem1, ssem0, ssem1, isem0, isem1):
  c = lax.axis_index("c")
  s = lax.axis_index("s")
  r0 = s * ROWS_PER_TILE
  f0 = s * FPT
  srcb = [src_b0, src_b1]
  dstb = [dst_b0, dst_b1]
  dsts = [dsts0, dsts1]
  rowsb = [rows_b0, rows_b1]
  gsem = [gsem0, gsem1]
  ssem = [ssem0, ssem1]
  isem = [isem0, isem1]

  pltpu.sync_copy(zf_hbm.at[pl.ds(r0, ROWS_PER_TILE)],
                  acc.at[pl.ds(r0, ROWS_PER_TILE)])
  pltpu.sync_copy(zw_hbm.at[pl.ds(f0, FPT)], waccf.at[pl.ds(f0, FPT)])
  pltpu.sync_copy(batchp_hbm, batch_p)
  for k in range(2 * EB // L):
    ones_b[pl.ds(k * L, L)] = jnp.ones((L,), jnp.float32)
  plsc.subcore_barrier()

  ebase = (c * NS + s) * TILE_EDGES

  def issue_idx(i, b):
    off = ebase + i * EB
    pltpu.async_copy(src_hbm.at[pl.ds(off, EB)], srcb[b], isem[b])
    pltpu.async_copy(dst_hbm.at[pl.ds(off, EB)], dstb[b], isem[b])

  def wait_idx(b):
    pltpu.make_async_copy(src_hbm.at[pl.ds(0, EB)], srcb[b], isem[b]).wait()
    pltpu.make_async_copy(dst_hbm.at[pl.ds(0, EB)], dstb[b], isem[b]).wait()

  def issue_gather(b):
    pltpu.async_copy(x_hbm.at[srcb[b]], rowsb[b], gsem[b])

  def wait_gather(b):
    pltpu.make_async_copy(x_hbm.at[srcb[b]], rowsb[b], gsem[b]).wait()

  def issue_scatter(b):
    pltpu.async_copy(rowsb[b], acc.at[dsts[b]], ssem[b], add=True)

  def wait_scatter(b):
    pltpu.make_async_copy(rowsb[b], acc.at[dsts[b]], ssem[b]).wait()

  def steps(i, b, first):
    wait_gather(b)
    for k in range(EB // L):
      sv = srcb[b][pl.ds(k * L, L)]
      dv = dstb[b][pl.ds(k * L, L)]
      dsts[b][pl.ds(k * L, L)] = dv
      wv = plsc.load_gather(batch_p, [dv >> 3])
      gv = (wv >> ((dv & 7) * 4)) & 15
      fi_b[pl.ds(b * EB + k * L, L)] = sv * G + gv
    issue_scatter(b)
    if b == 1:
      pltpu.sync_copy(ones_b, waccf.at[fi_b], add=True)
    wait_idx(1 - b)
    if not first:
      wait_scatter(1 - b)
    issue_gather(1 - b)
    issue_idx(i + 2, b)

  issue_idx(0, 0)
  wait_idx(0)
  issue_gather(0)
  issue_idx(1, 1)
  steps(0, 0, True)
  steps(1, 1, False)

  def pair_body(p, carry):
    i0 = 2 * p + 2
    steps(i0, 0, False)
    steps(i0 + 1, 1, False)
    return carry

  lax.fori_loop(0, (TILE_EDGES // EB - 2) // 2, pair_body, 0)

  wait_scatter(1)
  wait_gather(0)
  wait_idx(1)
  plsc.subcore_barrier()
  pltpu.sync_copy(acc.at[pl.ds(r0, ROWS_PER_TILE)],
                  agg_hbm.at[pl.ds(c * N_PAD + r0, ROWS_PER_TILE)])
  pltpu.sync_copy(waccf.at[pl.ds(f0, FPT)],
                  w_hbm.at[pl.ds(c * FLAT_W + f0, FPT)])


@jax.jit
def _sc_agg(x_pad, src_pad, dst_pad, batch_packed, zf, zwf):
  mesh = plsc.VectorSubcoreMesh(core_axis_name="c", subcore_axis_name="s",
                                num_cores=NC, num_subcores=NS)
  return pl.kernel(
      _sc_agg_body,
      out_type=(jax.ShapeDtypeStruct((NC * N_PAD, D), jnp.float32),
                jax.ShapeDtypeStruct((NC * FLAT_W,), jnp.float32)),
      mesh=mesh,
      compiler_params=pltpu.CompilerParams(needs_layout_passes=False),
      scratch_types=[
          pltpu.VMEM_SHARED((N_PAD, D), jnp.float32),
          pltpu.VMEM_SHARED((FLAT_W,), jnp.float32),
          pltpu.VMEM((N_PAD // 8,), jnp.int32),
          pltpu.VMEM((EB,), jnp.int32),
          pltpu.VMEM((EB,), jnp.int32),
          pltpu.VMEM((EB,), jnp.int32),
          pltpu.VMEM((EB,), jnp.int32),
          pltpu.VMEM((EB,), jnp.int32),
          pltpu.VMEM((EB,), jnp.int32),
          pltpu.VMEM((2 * EB,), jnp.int32),
          pltpu.VMEM((2 * EB,), jnp.float32),
          pltpu.VMEM((EB, D), jnp.float32),
          pltpu.VMEM((EB, D), jnp.float32),
          pltpu.SemaphoreType.DMA,
          pltpu.SemaphoreType.DMA,
          pltpu.SemaphoreType.DMA,
          pltpu.SemaphoreType.DMA,
          pltpu.SemaphoreType.DMA,
          pltpu.SemaphoreType.DMA,
      ],
  )(x_pad, src_pad, dst_pad, batch_packed, zf, zwf)


def _tc_body(agg_ref, w_ref, x_ref, batch_ref,
             w1r_ref, w1o_ref, b1_ref, w2r_ref, w2o_ref, b2_ref,
             wfc_ref, bfc_ref, out_ref, a_acc, b_acc, c_acc):
  i = pl.program_id(0)

  @pl.when(i == 0)
  def _init():
    a_acc[...] = jnp.zeros_like(a_acc)
    b_acc[...] = jnp.zeros_like(b_acc)
    c_acc[...] = jnp.zeros_like(c_acc)

  agg = agg_ref[0] + agg_ref[1]
  h = jnp.dot(agg, w1r_ref[...], preferred_element_type=jnp.float32)
  h += jnp.dot(x_ref[...], w1o_ref[...], preferred_element_type=jnp.float32)
  h = jnp.maximum(h + b1_ref[...], 0.0)

  rows = i * BR + lax.broadcasted_iota(jnp.int32, (BR, 1), 0)
  wm = jnp.where(rows < N, w_ref[0] + w_ref[1], 0.0)
  m = (batch_ref[...] ==
       lax.broadcasted_iota(jnp.int32, (BR, G), 1)).astype(jnp.float32)

  cdims = (((0,), (0,)), ((), ()))
  a_acc[...] += lax.dot_general(wm, h, cdims,
                                preferred_element_type=jnp.float32)
  b_acc[...] += lax.dot_general(m, h, cdims,
                                preferred_element_type=jnp.float32)
  c_acc[...] += lax.dot_general(m, jnp.ones((BR, D), jnp.float32), cdims,
                                preferred_element_type=jnp.float32)

  @pl.when(i == NBLK - 1)
  def _final():
    num = jnp.dot(a_acc[...], w2r_ref[...], preferred_element_type=jnp.float32)
    num += jnp.dot(b_acc[...], w2o_ref[...], preferred_element_type=jnp.float32)
    c = c_acc[...]
    pooled = (num + c * b2_ref[...]) / jnp.maximum(c, 1.0)
    out = jnp.dot(pooled, wfc_ref[...], preferred_element_type=jnp.float32)
    out_ref[...] = out + bfc_ref[...]


@jax.jit
def _tc_dense(agg_p, w_p, x_pad, batch_tc, w1rT, w1oT, b1, w2rT, w2oT, b2,
              wfcT, bfc):
  full = lambda shape: pl.BlockSpec(shape, lambda i: (0,) * len(shape))
  return pl.pallas_call(
      _tc_body,
      grid=(NBLK,),
      in_specs=[
          pl.BlockSpec((NC, BR, D), lambda i: (0, i, 0)),
          pl.BlockSpec((NC, BR, G), lambda i: (0, i, 0)),
          pl.BlockSpec((BR, D), lambda i: (i, 0)),
          pl.BlockSpec((BR, 1), lambda i: (i, 0)),
          full((D, D)), full((D, D)), full((1, D)),
          full((D, D)), full((D, D)), full((1, D)),
          full((D, FC_OUT)), full((1, FC_OUT)),
      ],
      out_specs=pl.BlockSpec((G, FC_OUT), lambda i: (0, 0)),
      out_shape=jax.ShapeDtypeStruct((G, FC_OUT), jnp.float32),
      scratch_shapes=[
          pltpu.VMEM((G, D), jnp.float32),
          pltpu.VMEM((G, D), jnp.float32),
          pltpu.VMEM((G, D), jnp.float32),
      ],
  )(agg_p, w_p, x_pad, batch_tc, w1rT, w1oT, b1, w2rT, w2oT, b2, wfcT, bfc)


def kernel(x, edge_index, batch, W1_rel, W1_root, b1, W2_rel, W2_root, b2,
           Wfc, bfc):
  src = edge_index[0].astype(jnp.int32)
  dst = edge_index[1].astype(jnp.int32)
  batch = batch.astype(jnp.int32)

  x_pad = jnp.zeros((N_PAD, D), jnp.float32).at[:N].set(x)
  pad_idx = N + (jnp.arange(E_PAD + 2 * EB - E, dtype=jnp.int32)
                 % (N_PAD - N))
  src_pad = jnp.concatenate([src, pad_idx])
  dst_pad = jnp.concatenate([dst, pad_idx])
  batch_sc = jnp.zeros((N_PAD,), jnp.int32).at[:N].set(batch)
  batch_packed = jnp.sum(
      batch_sc.reshape(-1, 8) << (4 * jnp.arange(8, dtype=jnp.int32))[None, :],
      axis=1, dtype=jnp.int32)
  batch_tc = jnp.full((N_PAD, 1), G, jnp.int32).at[:N, 0].set(batch)
  zf = jnp.zeros((N_PAD, D), jnp.float32)
  zwf = jnp.zeros((FLAT_W,), jnp.float32)

  agg_f, w_f = _sc_agg(x_pad, src_pad, dst_pad, batch_packed, zf, zwf)
  agg_p = agg_f.reshape(NC, N_PAD, D)
  w_p = w_f.reshape(NC, N_PAD, G)
  out = _tc_dense(agg_p, w_p, x_pad, batch_tc,
                  W1_rel.T, W1_root.T, b1[None], W2_rel.T, W2_root.T,
                  b2[None], Wfc.T, bfc[None])
  return out[None]


# speedup vs baseline: 17.7659x; 1.2923x over previous
import jax
import jax.numpy as jnp
from jax import lax
from jax.experimental import pallas as pl
from jax.experimental.pallas import tpu as pltpu
from jax.experimental.pallas import tpu_sc as plsc

N = 10000
E = 320000
D = 128
G = 16
FC_OUT = 2048

NC, NS, L = 2, 16, 16
NW = NC * NS
N_PAD = 10112
EB = 128
TILE_EDGES = 10240
E_PAD = TILE_EDGES * NW
ROWS_PER_TILE = N_PAD // NS
FLAT_W = N_PAD * G
FPT = FLAT_W // NS

BR = 1264
NBLK = N_PAD // BR


def _sc_agg_body(x_hbm, src_hbm, dst_hbm, batchp_hbm, zf_hbm, zw_hbm,
                 agg_hbm, w_hbm,
                 acc, waccf, batch_p,
                 src_b0, src_b1, dst_b0, dst_b1, dsts0, dsts1,
                 fi0, fi1, ones_b, rows_b0, rows_b1,
                 gsem0, gsem1, ssem0, ssem1, isem0, isem1, wsem0, wsem1):
  c = lax.axis_index("c")
  s = lax.axis_index("s")
  r0 = s * ROWS_PER_TILE
  f0 = s * FPT
  srcb = [src_b0, src_b1]
  dstb = [dst_b0, dst_b1]
  dsts = [dsts0, dsts1]
  fib = [fi0, fi1]
  rowsb = [rows_b0, rows_b1]
  gsem = [gsem0, gsem1]
  ssem = [ssem0, ssem1]
  isem = [isem0, isem1]
  wsem = [wsem0, wsem1]

  pltpu.sync_copy(zf_hbm.at[pl.ds(r0, ROWS_PER_TILE)],
                  acc.at[pl.ds(r0, ROWS_PER_TILE)])
  pltpu.sync_copy(zw_hbm.at[pl.ds(f0, FPT)], waccf.at[pl.ds(f0, FPT)])
  pltpu.sync_copy(batchp_hbm, batch_p)
  for k in range(EB // L):
    ones_b[pl.ds(k * L, L)] = jnp.ones((L,), jnp.float32)
  plsc.subcore_barrier()

  ebase = (c * NS + s) * TILE_EDGES

  def issue_idx(i, b):
    off = ebase + i * EB
    pltpu.async_copy(src_hbm.at[pl.ds(off, EB)], srcb[b], isem[b])
    pltpu.async_copy(dst_hbm.at[pl.ds(off, EB)], dstb[b], isem[b])

  def wait_idx(b):
    pltpu.make_async_copy(src_hbm.at[pl.ds(0, EB)], srcb[b], isem[b]).wait()
    pltpu.make_async_copy(dst_hbm.at[pl.ds(0, EB)], dstb[b], isem[b]).wait()

  def issue_gather(b):
    pltpu.async_copy(x_hbm.at[srcb[b]], rowsb[b], gsem[b])

  def wait_gather(b):
    pltpu.make_async_copy(x_hbm.at[srcb[b]], rowsb[b], gsem[b]).wait()

  def issue_scatter(b):
    pltpu.async_copy(rowsb[b], acc.at[dsts[b]], ssem[b], add=True)

  def wait_scatter(b):
    pltpu.make_async_copy(rowsb[b], acc.at[dsts[b]], ssem[b]).wait()

  def issue_wflush(b):
    pltpu.async_copy(ones_b, waccf.at[fib[b]], wsem[b], add=True)

  def wait_wflush(b):
    pltpu.make_async_copy(ones_b, waccf.at[fib[b]], wsem[b]).wait()

  def steps(i, b, first_flush, first_scatter):
    wait_gather(b)
    if not first_flush:
      wait_wflush(b)
    for k in range(EB // L):
      sv = srcb[b][pl.ds(k * L, L)]
      dv = dstb[b][pl.ds(k * L, L)]
      dsts[b][pl.ds(k * L, L)] = dv
      wv = plsc.load_gather(batch_p, [dv >> 3])
      gv = (wv >> ((dv & 7) * 4)) & 15
      fib[b][pl.ds(k * L, L)] = sv * G + gv
    issue_scatter(b)
    issue_wflush(b)
    wait_idx(1 - b)
    if not first_scatter:
      wait_scatter(1 - b)
    issue_gather(1 - b)
    issue_idx(i + 2, b)

  issue_idx(0, 0)
  wait_idx(0)
  issue_gather(0)
  issue_idx(1, 1)
  steps(0, 0, True, True)
  steps(1, 1, True, False)

  def pair_body(p, carry):
    i0 = 2 * p + 2
    steps(i0, 0, False, False)
    steps(i0 + 1, 1, False, False)
    return carry

  lax.fori_loop(0, (TILE_EDGES // EB - 2) // 2, pair_body, 0)

  wait_scatter(1)
  wait_wflush(0)
  wait_wflush(1)
  wait_gather(0)
  wait_idx(1)
  plsc.subcore_barrier()
  pltpu.sync_copy(acc.at[pl.ds(r0, ROWS_PER_TILE)],
                  agg_hbm.at[pl.ds(c * N_PAD + r0, ROWS_PER_TILE)])
  pltpu.sync_copy(waccf.at[pl.ds(f0, FPT)],
                  w_hbm.at[pl.ds(c * FLAT_W + f0, FPT)])


@jax.jit
def _sc_agg(x_pad, src_pad, dst_pad, batch_packed, zf, zwf):
  mesh = plsc.VectorSubcoreMesh(core_axis_name="c", subcore_axis_name="s",
                                num_cores=NC, num_subcores=NS)
  return pl.kernel(
      _sc_agg_body,
      out_type=(jax.ShapeDtypeStruct((NC * N_PAD, D), jnp.float32),
                jax.ShapeDtypeStruct((NC * FLAT_W,), jnp.float32)),
      mesh=mesh,
      compiler_params=pltpu.CompilerParams(needs_layout_passes=False),
      scratch_types=[
          pltpu.VMEM_SHARED((N_PAD, D), jnp.float32),
          pltpu.VMEM_SHARED((FLAT_W,), jnp.float32),
          pltpu.VMEM((N_PAD // 8,), jnp.int32),
          pltpu.VMEM((EB,), jnp.int32),
          pltpu.VMEM((EB,), jnp.int32),
          pltpu.VMEM((EB,), jnp.int32),
          pltpu.VMEM((EB,), jnp.int32),
          pltpu.VMEM((EB,), jnp.int32),
          pltpu.VMEM((EB,), jnp.int32),
          pltpu.VMEM((EB,), jnp.int32),
          pltpu.VMEM((EB,), jnp.int32),
          pltpu.VMEM((EB,), jnp.float32),
          pltpu.VMEM((EB, D), jnp.float32),
          pltpu.VMEM((EB, D), jnp.float32),
          pltpu.SemaphoreType.DMA,
          pltpu.SemaphoreType.DMA,
          pltpu.SemaphoreType.DMA,
          pltpu.SemaphoreType.DMA,
          pltpu.SemaphoreType.DMA,
          pltpu.SemaphoreType.DMA,
          pltpu.SemaphoreType.DMA,
          pltpu.SemaphoreType.DMA,
      ],
  )(x_pad, src_pad, dst_pad, batch_packed, zf, zwf)


def _tc_body(agg_ref, w_ref, x_ref, batch_ref,
             w1r_ref, w1o_ref, b1_ref, w2r_ref, w2o_ref, b2_ref,
             wfc_ref, bfc_ref, out_ref, a_acc, b_acc, c_acc):
  i = pl.program_id(0)

  @pl.when(i == 0)
  def _init():
    a_acc[...] = jnp.zeros_like(a_acc)
    b_acc[...] = jnp.zeros_like(b_acc)
    c_acc[...] = jnp.zeros_like(c_acc)

  agg = agg_ref[0] + agg_ref[1]
  h = jnp.dot(agg, w1r_ref[...], preferred_element_type=jnp.float32)
  h += jnp.dot(x_ref[...], w1o_ref[...], preferred_element_type=jnp.float32)
  h = jnp.maximum(h + b1_ref[...], 0.0)

  rows = i * BR + lax.broadcasted_iota(jnp.int32, (BR, 1), 0)
  wm = jnp.where(rows < N, w_ref[0] + w_ref[1], 0.0)
  m = (batch_ref[...] ==
       lax.broadcasted_iota(jnp.int32, (BR, G), 1)).astype(jnp.float32)

  cdims = (((0,), (0,)), ((), ()))
  a_acc[...] += lax.dot_general(wm, h, cdims,
                                preferred_element_type=jnp.float32)
  b_acc[...] += lax.dot_general(m, h, cdims,
                                preferred_element_type=jnp.float32)
  c_acc[...] += lax.dot_general(m, jnp.ones((BR, D), jnp.float32), cdims,
                                preferred_element_type=jnp.float32)

  @pl.when(i == NBLK - 1)
  def _final():
    num = jnp.dot(a_acc[...], w2r_ref[...], preferred_element_type=jnp.float32)
    num += jnp.dot(b_acc[...], w2o_ref[...], preferred_element_type=jnp.float32)
    c = c_acc[...]
    pooled = (num + c * b2_ref[...]) / jnp.maximum(c, 1.0)
    out = jnp.dot(pooled, wfc_ref[...], preferred_element_type=jnp.float32)
    out_ref[...] = out + bfc_ref[...]


@jax.jit
def _tc_dense(agg_p, w_p, x_pad, batch_tc, w1rT, w1oT, b1, w2rT, w2oT, b2,
              wfcT, bfc):
  full = lambda shape: pl.BlockSpec(shape, lambda i: (0,) * len(shape))
  return pl.pallas_call(
      _tc_body,
      grid=(NBLK,),
      in_specs=[
          pl.BlockSpec((NC, BR, D), lambda i: (0, i, 0)),
          pl.BlockSpec((NC, BR, G), lambda i: (0, i, 0)),
          pl.BlockSpec((BR, D), lambda i: (i, 0)),
          pl.BlockSpec((BR, 1), lambda i: (i, 0)),
          full((D, D)), full((D, D)), full((1, D)),
          full((D, D)), full((D, D)), full((1, D)),
          full((D, FC_OUT)), full((1, FC_OUT)),
      ],
      out_specs=pl.BlockSpec((G, FC_OUT), lambda i: (0, 0)),
      out_shape=jax.ShapeDtypeStruct((G, FC_OUT), jnp.float32),
      scratch_shapes=[
          pltpu.VMEM((G, D), jnp.float32),
          pltpu.VMEM((G, D), jnp.float32),
          pltpu.VMEM((G, D), jnp.float32),
      ],
  )(agg_p, w_p, x_pad, batch_tc, w1rT, w1oT, b1, w2rT, w2oT, b2, wfcT, bfc)


def kernel(x, edge_index, batch, W1_rel, W1_root, b1, W2_rel, W2_root, b2,
           Wfc, bfc):
  src = edge_index[0].astype(jnp.int32)
  dst = edge_index[1].astype(jnp.int32)
  batch = batch.astype(jnp.int32)

  x_pad = jnp.zeros((N_PAD, D), jnp.float32).at[:N].set(x)
  pad_idx = N + (jnp.arange(E_PAD + 2 * EB - E, dtype=jnp.int32)
                 % (N_PAD - N))
  src_pad = jnp.concatenate([src, pad_idx])
  dst_pad = jnp.concatenate([dst, pad_idx])
  batch_sc = jnp.zeros((N_PAD,), jnp.int32).at[:N].set(batch)
  batch_packed = jnp.sum(
      batch_sc.reshape(-1, 8) << (4 * jnp.arange(8, dtype=jnp.int32))[None, :],
      axis=1, dtype=jnp.int32)
  batch_tc = jnp.full((N_PAD, 1), G, jnp.int32).at[:N, 0].set(batch)
  zf = jnp.zeros((N_PAD, D), jnp.float32)
  zwf = jnp.zeros((FLAT_W,), jnp.float32)

  agg_f, w_f = _sc_agg(x_pad, src_pad, dst_pad, batch_packed, zf, zwf)
  agg_p = agg_f.reshape(NC, N_PAD, D)
  w_p = w_f.reshape(NC, N_PAD, G)
  out = _tc_dense(agg_p, w_p, x_pad, batch_tc,
                  W1_rel.T, W1_root.T, b1[None], W2_rel.T, W2_root.T,
                  b2[None], Wfc.T, bfc[None])
  return out[None]


# speedup vs baseline: 21.6437x; 1.2183x over previous
import jax
import jax.numpy as jnp
from jax import lax
from jax.experimental import pallas as pl
from jax.experimental.pallas import tpu as pltpu
from jax.experimental.pallas import tpu_sc as plsc

N = 10000
E = 320000
D = 128
G = 16
FC_OUT = 2048

NC, NS, L = 2, 16, 16
NW = NC * NS
N_PAD = 10112
EB = 64
NBUF = 4
TILE_EDGES = 10240
E_PAD = TILE_EDGES * NW
ROWS_PER_TILE = N_PAD // NS
FLAT_W = N_PAD * G
FPT = FLAT_W // NS

BR = 1264
NBLK = N_PAD // BR


def _sc_agg_body(x_hbm, src_hbm, dst_hbm, batchp_hbm, zw_hbm,
                 agg_hbm, w_hbm,
                 acc, waccf, batch_p,
                 src_b0, src_b1, src_b2, src_b3,
                 dst_b0, dst_b1, dst_b2, dst_b3,
                 dsts0, dsts1, dsts2, dsts3,
                 fi0, fi1, fi2, fi3, ones_b,
                 rows_b0, rows_b1, rows_b2, rows_b3,
                 gsem0, gsem1, gsem2, gsem3,
                 ssem0, ssem1, ssem2, ssem3,
                 isem0, isem1, isem2, isem3,
                 wsem0, wsem1, wsem2, wsem3):
  c = lax.axis_index("c")
  s = lax.axis_index("s")
  r0 = s * ROWS_PER_TILE
  f0 = s * FPT
  srcb = [src_b0, src_b1, src_b2, src_b3]
  dstb = [dst_b0, dst_b1, dst_b2, dst_b3]
  dsts = [dsts0, dsts1, dsts2, dsts3]
  fib = [fi0, fi1, fi2, fi3]
  rowsb = [rows_b0, rows_b1, rows_b2, rows_b3]
  gsem = [gsem0, gsem1, gsem2, gsem3]
  ssem = [ssem0, ssem1, ssem2, ssem3]
  isem = [isem0, isem1, isem2, isem3]
  wsem = [wsem0, wsem1, wsem2, wsem3]

  zv = jnp.zeros((L,), jnp.float32)
  for r in range(EB):
    for k in range(D // L):
      rows_b0[r, pl.ds(k * L, L)] = zv
  nfull, rem = divmod(ROWS_PER_TILE, EB)
  for j in range(nfull):
    pltpu.sync_copy(rows_b0, acc.at[pl.ds(r0 + j * EB, EB)])
  if rem:
    pltpu.sync_copy(rows_b0.at[pl.ds(0, rem)],
                    acc.at[pl.ds(r0 + nfull * EB, rem)])
  pltpu.sync_copy(zw_hbm.at[pl.ds(f0, FPT)], waccf.at[pl.ds(f0, FPT)])
  pltpu.sync_copy(batchp_hbm, batch_p)
  for k in range(EB // L):
    ones_b[pl.ds(k * L, L)] = jnp.ones((L,), jnp.float32)
  plsc.subcore_barrier()

  ebase = (c * NS + s) * TILE_EDGES

  def issue_idx(i, b):
    off = ebase + i * EB
    pltpu.async_copy(src_hbm.at[pl.ds(off, EB)], srcb[b], isem[b])
    pltpu.async_copy(dst_hbm.at[pl.ds(off, EB)], dstb[b], isem[b])

  def wait_idx(b):
    pltpu.make_async_copy(src_hbm.at[pl.ds(0, EB)], srcb[b], isem[b]).wait()
    pltpu.make_async_copy(dst_hbm.at[pl.ds(0, EB)], dstb[b], isem[b]).wait()

  def issue_gather(b):
    pltpu.async_copy(x_hbm.at[srcb[b]], rowsb[b], gsem[b])

  def wait_gather(b):
    pltpu.make_async_copy(x_hbm.at[srcb[b]], rowsb[b], gsem[b]).wait()

  def issue_scatter(b):
    pltpu.async_copy(rowsb[b], acc.at[dsts[b]], ssem[b], add=True)

  def wait_scatter(b):
    pltpu.make_async_copy(rowsb[b], acc.at[dsts[b]], ssem[b]).wait()

  def issue_wflush(b):
    pltpu.async_copy(ones_b, waccf.at[fib[b]], wsem[b], add=True)

  def wait_wflush(b):
    pltpu.make_async_copy(ones_b, waccf.at[fib[b]], wsem[b]).wait()

  def steps(i, b, wf, sc):
    bn = (b + NBUF - 1) % NBUF
    wait_gather(b)
    if wf:
      wait_wflush(b)
    for k in range(EB // L):
      sv = srcb[b][pl.ds(k * L, L)]
      dv = dstb[b][pl.ds(k * L, L)]
      dsts[b][pl.ds(k * L, L)] = dv
      wv = plsc.load_gather(batch_p, [dv >> 3])
      gv = (wv >> ((dv & 7) * 4)) & 15
      fib[b][pl.ds(k * L, L)] = sv * G + gv
    issue_scatter(b)
    issue_wflush(b)
    wait_idx(bn)
    if sc:
      wait_scatter(bn)
    issue_gather(bn)
    issue_idx(i + NBUF, b)

  for b in range(NBUF - 1):
    issue_idx(b, b)
    wait_idx(b)
    issue_gather(b)
  issue_idx(NBUF - 1, NBUF - 1)
  steps(0, 0, False, False)
  for b in range(1, NBUF):
    steps(b, b, False, True)

  def quad_body(q, carry):
    i0 = NBUF * q + NBUF
    for b in range(NBUF):
      steps(i0 + b, b, True, True)
    return carry

  lax.fori_loop(0, (TILE_EDGES // EB - NBUF) // NBUF, quad_body, 0)

  wait_scatter(NBUF - 1)
  for b in range(NBUF):
    wait_wflush(b)
  for b in range(NBUF - 1):
    wait_gather(b)
  wait_idx(NBUF - 1)
  plsc.subcore_barrier()
  pltpu.sync_copy(acc.at[pl.ds(r0, ROWS_PER_TILE)],
                  agg_hbm.at[pl.ds(c * N_PAD + r0, ROWS_PER_TILE)])
  pltpu.sync_copy(waccf.at[pl.ds(f0, FPT)],
                  w_hbm.at[pl.ds(c * FLAT_W + f0, FPT)])


@jax.jit
def _sc_agg(x_pad, src_pad, dst_pad, batch_packed, zwf):
  mesh = plsc.VectorSubcoreMesh(core_axis_name="c", subcore_axis_name="s",
                                num_cores=NC, num_subcores=NS)
  idx_bufs = [pltpu.VMEM((EB,), jnp.int32) for _ in range(4 * NBUF)]
  fi_bufs = [pltpu.VMEM((EB,), jnp.int32) for _ in range(NBUF)]
  row_bufs = [pltpu.VMEM((EB, D), jnp.float32) for _ in range(NBUF)]
  sems = [pltpu.SemaphoreType.DMA for _ in range(4 * NBUF)]
  return pl.kernel(
      _sc_agg_body,
      out_type=(jax.ShapeDtypeStruct((NC * N_PAD, D), jnp.float32),
                jax.ShapeDtypeStruct((NC * FLAT_W,), jnp.float32)),
      mesh=mesh,
      compiler_params=pltpu.CompilerParams(needs_layout_passes=False),
      scratch_types=[
          pltpu.VMEM_SHARED((N_PAD, D), jnp.float32),
          pltpu.VMEM_SHARED((FLAT_W,), jnp.float32),
          pltpu.VMEM((N_PAD // 8,), jnp.int32),
      ] + idx_bufs[:3 * NBUF] + fi_bufs + [
          pltpu.VMEM((EB,), jnp.float32),
      ] + row_bufs + sems,
  )(x_pad, src_pad, dst_pad, batch_packed, zwf)


def _tc_body(agg_ref, w_ref, x_ref, batch_ref,
             w1r_ref, w1o_ref, b1_ref, w2r_ref, w2o_ref, b2_ref,
             wfc_ref, bfc_ref, out_ref, a_acc, b_acc, c_acc):
  i = pl.program_id(0)

  @pl.when(i == 0)
  def _init():
    a_acc[...] = jnp.zeros_like(a_acc)
    b_acc[...] = jnp.zeros_like(b_acc)
    c_acc[...] = jnp.zeros_like(c_acc)

  agg = agg_ref[0] + agg_ref[1]
  h = jnp.dot(agg, w1r_ref[...], preferred_element_type=jnp.float32)
  h += jnp.dot(x_ref[...], w1o_ref[...], preferred_element_type=jnp.float32)
  h = jnp.maximum(h + b1_ref[...], 0.0)

  rows = i * BR + lax.broadcasted_iota(jnp.int32, (BR, 1), 0)
  wm = jnp.where(rows < N, w_ref[0] + w_ref[1], 0.0)
  m = (batch_ref[...] ==
       lax.broadcasted_iota(jnp.int32, (BR, G), 1)).astype(jnp.float32)

  cdims = (((0,), (0,)), ((), ()))
  a_acc[...] += lax.dot_general(wm, h, cdims,
                                preferred_element_type=jnp.float32)
  b_acc[...] += lax.dot_general(m, h, cdims,
                                preferred_element_type=jnp.float32)
  c_acc[...] += lax.dot_general(m, jnp.ones((BR, D), jnp.float32), cdims,
                                preferred_element_type=jnp.float32)

  @pl.when(i == NBLK - 1)
  def _final():
    num = jnp.dot(a_acc[...], w2r_ref[...], preferred_element_type=jnp.float32)
    num += jnp.dot(b_acc[...], w2o_ref[...], preferred_element_type=jnp.float32)
    c = c_acc[...]
    pooled = (num + c * b2_ref[...]) / jnp.maximum(c, 1.0)
    out = jnp.dot(pooled, wfc_ref[...], preferred_element_type=jnp.float32)
    out_ref[...] = out + bfc_ref[...]


@jax.jit
def _tc_dense(agg_p, w_p, x_pad, batch_tc, w1rT, w1oT, b1, w2rT, w2oT, b2,
              wfcT, bfc):
  full = lambda shape: pl.BlockSpec(shape, lambda i: (0,) * len(shape))
  return pl.pallas_call(
      _tc_body,
      grid=(NBLK,),
      in_specs=[
          pl.BlockSpec((NC, BR, D), lambda i: (0, i, 0)),
          pl.BlockSpec((NC, BR, G), lambda i: (0, i, 0)),
          pl.BlockSpec((BR, D), lambda i: (i, 0)),
          pl.BlockSpec((BR, 1), lambda i: (i, 0)),
          full((D, D)), full((D, D)), full((1, D)),
          full((D, D)), full((D, D)), full((1, D)),
          full((D, FC_OUT)), full((1, FC_OUT)),
      ],
      out_specs=pl.BlockSpec((G, FC_OUT), lambda i: (0, 0)),
      out_shape=jax.ShapeDtypeStruct((G, FC_OUT), jnp.float32),
      scratch_shapes=[
          pltpu.VMEM((G, D), jnp.float32),
          pltpu.VMEM((G, D), jnp.float32),
          pltpu.VMEM((G, D), jnp.float32),
      ],
  )(agg_p, w_p, x_pad, batch_tc, w1rT, w1oT, b1, w2rT, w2oT, b2, wfcT, bfc)


def kernel(x, edge_index, batch, W1_rel, W1_root, b1, W2_rel, W2_root, b2,
           Wfc, bfc):
  src = edge_index[0].astype(jnp.int32)
  dst = edge_index[1].astype(jnp.int32)
  batch = batch.astype(jnp.int32)

  x_pad = jnp.zeros((N_PAD, D), jnp.float32).at[:N].set(x)
  pad_idx = N + (jnp.arange(E_PAD + NBUF * EB - E, dtype=jnp.int32)
                 % (N_PAD - N))
  src_pad = jnp.concatenate([src, pad_idx])
  dst_pad = jnp.concatenate([dst, pad_idx])
  batch_sc = jnp.zeros((N_PAD,), jnp.int32).at[:N].set(batch)
  batch_packed = jnp.sum(
      batch_sc.reshape(-1, 8) << (4 * jnp.arange(8, dtype=jnp.int32))[None, :],
      axis=1, dtype=jnp.int32)
  batch_tc = jnp.full((N_PAD, 1), G, jnp.int32).at[:N, 0].set(batch)
  zwf = jnp.zeros((FLAT_W,), jnp.float32)

  agg_f, w_f = _sc_agg(x_pad, src_pad, dst_pad, batch_packed, zwf)
  agg_p = agg_f.reshape(NC, N_PAD, D)
  w_p = w_f.reshape(NC, N_PAD, G)
  out = _tc_dense(agg_p, w_p, x_pad, batch_tc,
                  W1_rel.T, W1_root.T, b1[None], W2_rel.T, W2_root.T,
                  b2[None], Wfc.T, bfc[None])
  return out[None]


# speedup vs baseline: 21.6599x; 1.0007x over previous
import jax
import jax.numpy as jnp
from jax import lax
from jax.experimental import pallas as pl
from jax.experimental.pallas import tpu as pltpu
from jax.experimental.pallas import tpu_sc as plsc

N = 10000
E = 320000
D = 128
G = 16
FC_OUT = 2048

NC, NS, L = 2, 16, 16
NW = NC * NS
N_PAD = 10112
EB = 64
NBUF = 4
TILE_EDGES = 10240
E_PAD = TILE_EDGES * NW
ROWS_PER_TILE = N_PAD // NS
FLAT_W = N_PAD * G
FPT = FLAT_W // NS

BR = 1264
NBLK = N_PAD // BR


def _sc_agg_body(x_hbm, src_hbm, dst_hbm, batchp_hbm, zw_hbm,
                 agg_hbm, w_hbm,
                 acc, waccf, batch_p,
                 src_b0, src_b1, src_b2, src_b3,
                 dst_b0, dst_b1, dst_b2, dst_b3,
                 dsts0, dsts1, dsts2, dsts3,
                 fi0, fi1, fi2, fi3, ones_b,
                 rows_b0, rows_b1, rows_b2, rows_b3,
                 gsem0, gsem1, gsem2, gsem3,
                 ssem0, ssem1, ssem2, ssem3,
                 isem0, isem1, isem2, isem3,
                 wsem0, wsem1, wsem2, wsem3):
  c = lax.axis_index("c")
  s = lax.axis_index("s")
  r0 = s * ROWS_PER_TILE
  f0 = s * FPT
  srcb = [src_b0, src_b1, src_b2, src_b3]
  dstb = [dst_b0, dst_b1, dst_b2, dst_b3]
  dsts = [dsts0, dsts1, dsts2, dsts3]
  fib = [fi0, fi1, fi2, fi3]
  rowsb = [rows_b0, rows_b1, rows_b2, rows_b3]
  gsem = [gsem0, gsem1, gsem2, gsem3]
  ssem = [ssem0, ssem1, ssem2, ssem3]
  isem = [isem0, isem1, isem2, isem3]
  wsem = [wsem0, wsem1, wsem2, wsem3]

  zv = jnp.zeros((L,), jnp.float32)
  for r in range(EB):
    for k in range(D // L):
      rows_b0[r, pl.ds(k * L, L)] = zv
  nfull, rem = divmod(ROWS_PER_TILE, EB)
  for j in range(nfull):
    pltpu.sync_copy(rows_b0, acc.at[pl.ds(r0 + j * EB, EB)])
  if rem:
    pltpu.sync_copy(rows_b0.at[pl.ds(0, rem)],
                    acc.at[pl.ds(r0 + nfull * EB, rem)])
  pltpu.sync_copy(zw_hbm.at[pl.ds(f0, FPT)], waccf.at[pl.ds(f0, FPT)])
  pltpu.sync_copy(batchp_hbm, batch_p)
  for k in range(EB // L):
    ones_b[pl.ds(k * L, L)] = jnp.ones((L,), jnp.float32)
  plsc.subcore_barrier()

  ebase = (c * NS + s) * TILE_EDGES

  def issue_idx(i, b):
    off = ebase + i * EB
    pltpu.async_copy(src_hbm.at[pl.ds(off, EB)], srcb[b], isem[b])
    pltpu.async_copy(dst_hbm.at[pl.ds(off, EB)], dstb[b], isem[b])

  def wait_idx(b):
    pltpu.make_async_copy(src_hbm.at[pl.ds(0, EB)], srcb[b], isem[b]).wait()
    pltpu.make_async_copy(dst_hbm.at[pl.ds(0, EB)], dstb[b], isem[b]).wait()

  def issue_gather(b):
    pltpu.async_copy(x_hbm.at[srcb[b]], rowsb[b], gsem[b])

  def wait_gather(b):
    pltpu.make_async_copy(x_hbm.at[srcb[b]], rowsb[b], gsem[b]).wait()

  def issue_scatter(b):
    pltpu.async_copy(rowsb[b], acc.at[dsts[b]], ssem[b], add=True)

  def wait_scatter(b):
    pltpu.make_async_copy(rowsb[b], acc.at[dsts[b]], ssem[b]).wait()

  def issue_wflush(b):
    pltpu.async_copy(ones_b, waccf.at[fib[b]], wsem[b], add=True)

  def wait_wflush(b):
    pltpu.make_async_copy(ones_b, waccf.at[fib[b]], wsem[b]).wait()

  def steps(i, b, wf, sc):
    bn = (b + NBUF - 1) % NBUF
    wait_gather(b)
    if wf:
      wait_wflush(b)
    for k in range(EB // L):
      sv = srcb[b][pl.ds(k * L, L)]
      dv = dstb[b][pl.ds(k * L, L)]
      dsts[b][pl.ds(k * L, L)] = dv
      wv = plsc.load_gather(batch_p, [dv >> 3])
      gv = (wv >> ((dv & 7) * 4)) & 15
      fib[b][pl.ds(k * L, L)] = sv * G + gv
    issue_scatter(b)
    issue_wflush(b)
    wait_idx(bn)
    if sc:
      wait_scatter(bn)
    issue_gather(bn)
    issue_idx(i + NBUF, b)

  for b in range(NBUF - 1):
    issue_idx(b, b)
    wait_idx(b)
    issue_gather(b)
  issue_idx(NBUF - 1, NBUF - 1)
  steps(0, 0, False, False)
  for b in range(1, NBUF):
    steps(b, b, False, True)

  def quad_body(q, carry):
    i0 = NBUF * q + NBUF
    for b in range(NBUF):
      steps(i0 + b, b, True, True)
    return carry

  lax.fori_loop(0, (TILE_EDGES // EB - NBUF) // NBUF, quad_body, 0)

  wait_scatter(NBUF - 1)
  for b in range(NBUF):
    wait_wflush(b)
  for b in range(NBUF - 1):
    wait_gather(b)
  wait_idx(NBUF - 1)
  plsc.subcore_barrier()
  pltpu.sync_copy(acc.at[pl.ds(r0, ROWS_PER_TILE)],
                  agg_hbm.at[pl.ds(c * N_PAD + r0, ROWS_PER_TILE)])
  pltpu.sync_copy(waccf.at[pl.ds(f0, FPT)],
                  w_hbm.at[pl.ds(c * FLAT_W + f0, FPT)])


def _sc_agg(x_pad, src_pad, dst_pad, batch_packed, zwf):
  mesh = plsc.VectorSubcoreMesh(core_axis_name="c", subcore_axis_name="s",
                                num_cores=NC, num_subcores=NS)
  idx_bufs = [pltpu.VMEM((EB,), jnp.int32) for _ in range(4 * NBUF)]
  fi_bufs = [pltpu.VMEM((EB,), jnp.int32) for _ in range(NBUF)]
  row_bufs = [pltpu.VMEM((EB, D), jnp.float32) for _ in range(NBUF)]
  sems = [pltpu.SemaphoreType.DMA for _ in range(4 * NBUF)]
  return pl.kernel(
      _sc_agg_body,
      out_type=(jax.ShapeDtypeStruct((NC * N_PAD, D), jnp.float32),
                jax.ShapeDtypeStruct((NC * FLAT_W,), jnp.float32)),
      mesh=mesh,
      compiler_params=pltpu.CompilerParams(needs_layout_passes=False),
      scratch_types=[
          pltpu.VMEM_SHARED((N_PAD, D), jnp.float32),
          pltpu.VMEM_SHARED((FLAT_W,), jnp.float32),
          pltpu.VMEM((N_PAD // 8,), jnp.int32),
      ] + idx_bufs[:3 * NBUF] + fi_bufs + [
          pltpu.VMEM((EB,), jnp.float32),
      ] + row_bufs + sems,
  )(x_pad, src_pad, dst_pad, batch_packed, zwf)


def _tc_body(agg_ref, w_ref, x_ref, batch_ref,
             w1r_ref, w1o_ref, b1_ref, w2r_ref, w2o_ref, b2_ref,
             wfc_ref, bfc_ref, out_ref, a_acc, b_acc, c_acc):
  i = pl.program_id(0)

  @pl.when(i == 0)
  def _init():
    a_acc[...] = jnp.zeros_like(a_acc)
    b_acc[...] = jnp.zeros_like(b_acc)
    c_acc[...] = jnp.zeros_like(c_acc)

  agg = agg_ref[0] + agg_ref[1]
  h = jnp.dot(agg, w1r_ref[...], preferred_element_type=jnp.float32)
  h += jnp.dot(x_ref[...], w1o_ref[...], preferred_element_type=jnp.float32)
  h = jnp.maximum(h + b1_ref[...], 0.0)

  rows = i * BR + lax.broadcasted_iota(jnp.int32, (BR, 1), 0)
  wm = jnp.where(rows < N, w_ref[0] + w_ref[1], 0.0)
  m = (batch_ref[...] ==
       lax.broadcasted_iota(jnp.int32, (BR, G), 1)).astype(jnp.float32)

  cdims = (((0,), (0,)), ((), ()))
  a_acc[...] += lax.dot_general(wm, h, cdims,
                                preferred_element_type=jnp.float32)
  b_acc[...] += lax.dot_general(m, h, cdims,
                                preferred_element_type=jnp.float32)
  c_acc[...] += lax.dot_general(m, jnp.ones((BR, D), jnp.float32), cdims,
                                preferred_element_type=jnp.float32)

  @pl.when(i == NBLK - 1)
  def _final():
    num = jnp.dot(a_acc[...], w2r_ref[...], preferred_element_type=jnp.float32)
    num += jnp.dot(b_acc[...], w2o_ref[...], preferred_element_type=jnp.float32)
    c = c_acc[...]
    pooled = (num + c * b2_ref[...]) / jnp.maximum(c, 1.0)
    out = jnp.dot(pooled, wfc_ref[...], preferred_element_type=jnp.float32)
    out_ref[...] = out + bfc_ref[...]


def _tc_dense(agg_p, w_p, x_pad, batch_tc, w1rT, w1oT, b1, w2rT, w2oT, b2,
              wfcT, bfc):
  full = lambda shape: pl.BlockSpec(shape, lambda i: (0,) * len(shape))
  return pl.pallas_call(
      _tc_body,
      grid=(NBLK,),
      in_specs=[
          pl.BlockSpec((NC, BR, D), lambda i: (0, i, 0)),
          pl.BlockSpec((NC, BR, G), lambda i: (0, i, 0)),
          pl.BlockSpec((BR, D), lambda i: (i, 0)),
          pl.BlockSpec((BR, 1), lambda i: (i, 0)),
          full((D, D)), full((D, D)), full((1, D)),
          full((D, D)), full((D, D)), full((1, D)),
          full((D, FC_OUT)), full((1, FC_OUT)),
      ],
      out_specs=pl.BlockSpec((G, FC_OUT), lambda i: (0, 0)),
      out_shape=jax.ShapeDtypeStruct((G, FC_OUT), jnp.float32),
      scratch_shapes=[
          pltpu.VMEM((G, D), jnp.float32),
          pltpu.VMEM((G, D), jnp.float32),
          pltpu.VMEM((G, D), jnp.float32),
      ],
  )(agg_p, w_p, x_pad, batch_tc, w1rT, w1oT, b1, w2rT, w2oT, b2, wfcT, bfc)


@jax.jit
def _impl(x, edge_index, batch, W1_rel, W1_root, b1, W2_rel, W2_root, b2,
          Wfc, bfc):
  src = edge_index[0].astype(jnp.int32)
  dst = edge_index[1].astype(jnp.int32)
  batch = batch.astype(jnp.int32)

  x_pad = jnp.zeros((N_PAD, D), jnp.float32).at[:N].set(x)
  pad_idx = N + (jnp.arange(E_PAD + NBUF * EB - E, dtype=jnp.int32)
                 % (N_PAD - N))
  src_pad = jnp.concatenate([src, pad_idx])
  dst_pad = jnp.concatenate([dst, pad_idx])
  batch_sc = jnp.zeros((N_PAD,), jnp.int32).at[:N].set(batch)
  batch_packed = jnp.sum(
      batch_sc.reshape(-1, 8) << (4 * jnp.arange(8, dtype=jnp.int32))[None, :],
      axis=1, dtype=jnp.int32)
  batch_tc = jnp.full((N_PAD, 1), G, jnp.int32).at[:N, 0].set(batch)
  zwf = jnp.zeros((FLAT_W,), jnp.float32)

  agg_f, w_f = _sc_agg(x_pad, src_pad, dst_pad, batch_packed, zwf)
  agg_p = agg_f.reshape(NC, N_PAD, D)
  w_p = w_f.reshape(NC, N_PAD, G)
  out = _tc_dense(agg_p, w_p, x_pad, batch_tc,
                  W1_rel.T, W1_root.T, b1[None], W2_rel.T, W2_root.T,
                  b2[None], Wfc.T, bfc[None])
  return out[None]


def kernel(x, edge_index, batch, W1_rel, W1_root, b1, W2_rel, W2_root, b2,
           Wfc, bfc):
  return _impl(x, edge_index, batch, W1_rel, W1_root, b1, W2_rel, W2_root,
               b2, Wfc, bfc)
